# baseline (speedup 1.0000x reference)
import functools

import jax
import jax.numpy as jnp
from jax import lax
from jax.experimental import pallas as pl
from jax.experimental.pallas import tpu as pltpu
from jax.experimental.pallas import tpu_sc as plsc

F32 = jnp.float32

_N = 10000
_D = 128
_H = 128
_E = 320000

_NC = 2
_NS = 16
_NW = _NC * _NS
_CHUNK = 128
_EW = 10240
_EPAD = _EW * _NW
_EROWS = _EPAD // _CHUNK
_NPAD = 10240

_AW_ROWS = _EW // _CHUNK
_AW0 = 80
_ANROWS = 10112
_ASUB = _ANROWS // _NS

_R = 1000
_GRID = _N // _R


def _make_deg_kernel():
    mesh = plsc.VectorSubcoreMesh(core_axis_name="c", subcore_axis_name="s")
    rows_per_sub = _EROWS // _NS

    @functools.partial(
        pl.kernel,
        mesh=mesh,
        out_type=jax.ShapeDtypeStruct((_NPAD,), F32),
        scratch_types=[
            pltpu.VMEM((rows_per_sub, _CHUNK), jnp.int32),
            pltpu.VMEM((_CHUNK,), F32),
            pltpu.VMEM((_NPAD // _NS,), F32),
            pltpu.VMEM_SHARED((_NPAD,), F32),
        ],
    )
    def deg_kernel(dst_hbm, deg_hbm, dst_v, ones_v, dbuf, deg_sh):
        c = lax.axis_index("c")
        s = lax.axis_index("s")
        for i in range(_CHUNK // 16):
            ones_v[pl.ds(i * 16, 16)] = jnp.full((16,), 1.0, F32)
        for i in range((_NPAD // _NS) // 16):
            dbuf[pl.ds(i * 16, 16)] = jnp.zeros((16,), F32)
        pltpu.sync_copy(dbuf, deg_sh.at[pl.ds(s * 640, 640)])
        plsc.subcore_barrier()

        @pl.when(c == 0)
        def _count():
            pltpu.sync_copy(dst_hbm.at[pl.ds(s * rows_per_sub, rows_per_sub), :], dst_v)

            def body(t, carry):
                pltpu.sync_copy(ones_v, deg_sh.at[dst_v.at[t]], add=True)
                return carry

            lax.fori_loop(0, rows_per_sub, body, 0)

        plsc.subcore_barrier()

        @pl.when(c == 0)
        def _finish():
            pltpu.sync_copy(deg_sh.at[pl.ds(s * 640, 640)], dbuf)
            pltpu.sync_copy(dbuf, deg_hbm.at[pl.ds(s * 640, 640)])

    return deg_kernel


def _make_agg_kernel():
    mesh = plsc.VectorSubcoreMesh(core_axis_name="c", subcore_axis_name="s")

    @functools.partial(
        pl.kernel,
        mesh=mesh,
        out_type=jax.ShapeDtypeStruct((_ANROWS, _H), F32),
        scratch_types=[
            pltpu.VMEM((_AW0, _CHUNK), jnp.int32),
            pltpu.VMEM((1, _CHUNK), jnp.int32),
            pltpu.VMEM((1, _CHUNK), jnp.int32),
            pltpu.VMEM((1, _CHUNK), jnp.int32),
            pltpu.VMEM((1, _CHUNK), jnp.int32),
            pltpu.VMEM((_CHUNK, _H), F32),
            pltpu.VMEM((_CHUNK, _H), F32),
            pltpu.VMEM_SHARED((_ANROWS, _H), F32),
            pltpu.SemaphoreType.DMA,
            pltpu.SemaphoreType.DMA,
        ],
    )
    def agg_kernel(g_hbm, packed_hbm, out_hbm,
                   pk_v, sr0, sr1, dr0, dr1, b0, b1, acc_sh, s0, s1):
        bufs = (b0, b1)
        sems = (s0, s1)
        srct = (sr0, sr1)
        dstt = (dr0, dr1)
        c = lax.axis_index("c")
        s = lax.axis_index("s")

        def unpack(t, r):
            for j in range(_CHUNK // 16):
                p = pk_v[t, pl.ds(j * 16, 16)]
                srct[r][0, pl.ds(j * 16, 16)] = lax.bitwise_and(
                    p, jnp.int32(0x3FFF))
                dstt[r][0, pl.ds(j * 16, 16)] = lax.shift_right_logical(p, 14)

        def run_edges(base, nrows):
            pltpu.sync_copy(
                packed_hbm.at[pl.ds(base, nrows), :], pk_v.at[pl.ds(0, nrows), :])
            unpack(0, 0)
            pltpu.async_copy(g_hbm.at[sr0.at[0]], b0, s0)

            def phase(t, p):
                np_ = (p + 1) % 2
                pltpu.make_async_copy(
                    g_hbm.at[srct[p].at[0]], bufs[p], sems[p]).wait()

                @pl.when(t + 1 < nrows)
                def _():
                    unpack(t + 1, np_)
                    pltpu.async_copy(
                        g_hbm.at[srct[np_].at[0]], bufs[np_], sems[np_])

                pltpu.sync_copy(bufs[p], acc_sh.at[dstt[p].at[0]], add=True)

            def body(j, carry):
                phase(2 * j, 0)
                phase(2 * j + 1, 1)
                return carry

            lax.fori_loop(0, nrows // 2, body, 0)

        @pl.when(c == 0)
        def _work():
            def zrow(i, carry):
                for j in range(_H // 16):
                    b0[i, pl.ds(j * 16, 16)] = jnp.zeros((16,), F32)
                return carry

            lax.fori_loop(0, _CHUNK, zrow, 0)
            for r0 in range(0, _ASUB, _CHUNK):
                sz = min(_CHUNK, _ASUB - r0)
                pltpu.sync_copy(
                    b0.at[pl.ds(0, sz), :],
                    acc_sh.at[pl.ds(s * _ASUB + r0, sz), :],
                )
            plsc.subcore_barrier()

            run_edges(s * 2 * _AW0, _AW0)
            run_edges(s * 2 * _AW0 + _AW0, _AW0)
            plsc.subcore_barrier()

            for r0 in range(0, _ASUB, _CHUNK):
                sz = min(_CHUNK, _ASUB - r0)
                row = s * _ASUB + r0
                pltpu.sync_copy(acc_sh.at[pl.ds(row, sz), :], b0.at[pl.ds(0, sz), :])
                pltpu.sync_copy(b0.at[pl.ds(0, sz), :], out_hbm.at[pl.ds(row, sz), :])

    return agg_kernel


def _dinv(deg_ref):
    return lax.rsqrt(deg_ref[...] + 1.0)


def _tc1_body(x_ref, w_ref, deg_ref, g_ref):
    g_ref[...] = (
        jnp.dot(x_ref[...], w_ref[...], preferred_element_type=F32) * _dinv(deg_ref)
    )


def _tc2_body(a0_ref, g_ref, deg_ref, b_ref, w_ref, g2_ref, h_ref):
    dinv = _dinv(deg_ref)
    a = a0_ref[...] + g_ref[...]
    h = jnp.maximum(a * dinv + b_ref[...], 0.0)
    h_ref[...] = h
    g2_ref[...] = jnp.dot(h, w_ref[...], preferred_element_type=F32) * dinv


def _tc3_body(a0_ref, g_ref, deg_ref, b_ref, res_ref, w_ref, g2_ref, h_ref):
    dinv = _dinv(deg_ref)
    a = a0_ref[...] + g_ref[...]
    h = jnp.maximum(a * dinv + b_ref[...] + res_ref[...], 0.0)
    h_ref[...] = h
    g2_ref[...] = jnp.dot(h, w_ref[...], preferred_element_type=F32) * dinv


def _tc4_body(a0_ref, g_ref, deg_ref, b_ref, res_ref, fcw_ref, fcb_ref, o_ref):
    dinv = _dinv(deg_ref)
    a = a0_ref[...] + g_ref[...]
    h = jnp.maximum(a * dinv + b_ref[...] + res_ref[...], 0.0)
    o = jnp.dot(h, fcw_ref[...], preferred_element_type=F32) + fcb_ref[...]
    o_ref[...] = jax.nn.sigmoid(o)


def _row_spec(i):
    return (i, 0)


def _full_spec(i):
    return (0, 0)


_ROWS = pl.BlockSpec((_R, _H), _row_spec)
_DINV = pl.BlockSpec((_R, 1), _row_spec)
_WFULL = pl.BlockSpec((_D, _H), _full_spec)
_BFULL = pl.BlockSpec((1, _H), _full_spec)


def _tc1(x, W, deg):
    return pl.pallas_call(
        _tc1_body,
        grid=(_GRID,),
        in_specs=[_ROWS, _WFULL, _DINV],
        out_specs=_ROWS,
        out_shape=jax.ShapeDtypeStruct((_N, _H), F32),
    )(x, W, deg)


def _tc2(acc, g, deg, b, W):
    return pl.pallas_call(
        _tc2_body,
        grid=(_GRID,),
        in_specs=[_ROWS, _ROWS, _DINV, _BFULL, _WFULL],
        out_specs=[_ROWS, _ROWS],
        out_shape=[
            jax.ShapeDtypeStruct((_N, _H), F32),
            jax.ShapeDtypeStruct((_N, _H), F32),
        ],
    )(acc, g, deg, b, W)


def _tc3(acc, g, deg, b, res, W):
    return pl.pallas_call(
        _tc3_body,
        grid=(_GRID,),
        in_specs=[_ROWS, _ROWS, _DINV, _BFULL, _ROWS, _WFULL],
        out_specs=[_ROWS, _ROWS],
        out_shape=[
            jax.ShapeDtypeStruct((_N, _H), F32),
            jax.ShapeDtypeStruct((_N, _H), F32),
        ],
    )(acc, g, deg, b, res, W)


def _tc4(acc, g, deg, b, res, fcW, fcb):
    return pl.pallas_call(
        _tc4_body,
        grid=(_GRID,),
        in_specs=[
            _ROWS,
            _ROWS,
            _DINV,
            _BFULL,
            _ROWS,
            pl.BlockSpec((_H, 1), _full_spec),
            pl.BlockSpec((1, 1), _full_spec),
        ],
        out_specs=pl.BlockSpec((_R, 1), _row_spec),
        out_shape=jax.ShapeDtypeStruct((_N, 1), F32),
    )(acc, g, deg, b, res, fcW, fcb)


_deg_call = _make_deg_kernel()
_agg_call = _make_agg_kernel()


def kernel(x, edge_index, W1, b1, W2, b2, W3, b3, fc_W, fc_b):
    pad = _EPAD - _E
    src = jnp.concatenate([edge_index[0], jnp.zeros((pad,), jnp.int32)])
    dst = jnp.concatenate([edge_index[1], jnp.full((pad,), _N, jnp.int32)])
    packed2d = jnp.bitwise_or(src, jnp.left_shift(dst, 14)).reshape(_EROWS, _CHUNK)
    dst2d_deg = dst.reshape(_EROWS, _CHUNK)

    deg = _deg_call(dst2d_deg)[:_N].reshape(_N, 1)

    b1r = b1.reshape(1, _H)
    b2r = b2.reshape(1, _H)
    b3r = b3.reshape(1, _H)
    fcbr = fc_b.reshape(1, 1)

    g1 = _tc1(x, W1, deg)
    acc1 = _agg_call(g1, packed2d)
    g2, h1 = _tc2(acc1, g1, deg, b1r, W2)
    acc2 = _agg_call(g2, packed2d)
    g3, h2 = _tc3(acc2, g2, deg, b2r, h1, W3)
    acc3 = _agg_call(g3, packed2d)
    out = _tc4(acc3, g3, deg, b3r, h2, fc_W, fcbr)
    return out


# speedup vs baseline: 1.4652x; 1.4652x over previous
import functools

import jax
import jax.numpy as jnp
from jax import lax
from jax.experimental import pallas as pl
from jax.experimental.pallas import tpu as pltpu
from jax.experimental.pallas import tpu_sc as plsc

F32 = jnp.float32

_N = 10000
_D = 128
_H = 128
_E = 320000

_NC = 2
_NS = 16
_NW = _NC * _NS
_CHUNK = 128
_EW = 10240
_EPAD = _EW * _NW
_EROWS = _EPAD // _CHUNK
_NPAD = 10240

_AW_ROWS = _EW // _CHUNK
_AW0 = 128
_AW1 = 32
_ANROWS = 10112
_ASUB = _ANROWS // _NS

_R = 1000
_GRID = _N // _R


def _make_deg_kernel():
    mesh = plsc.VectorSubcoreMesh(core_axis_name="c", subcore_axis_name="s")
    rows_per_sub = _EROWS // _NS

    @functools.partial(
        pl.kernel,
        mesh=mesh,
        out_type=jax.ShapeDtypeStruct((_NPAD,), F32),
        scratch_types=[
            pltpu.VMEM((rows_per_sub, _CHUNK), jnp.int32),
            pltpu.VMEM((_CHUNK,), F32),
            pltpu.VMEM((_NPAD // _NS,), F32),
            pltpu.VMEM_SHARED((_NPAD,), F32),
        ],
    )
    def deg_kernel(dst_hbm, deg_hbm, dst_v, ones_v, dbuf, deg_sh):
        c = lax.axis_index("c")
        s = lax.axis_index("s")
        for i in range(_CHUNK // 16):
            ones_v[pl.ds(i * 16, 16)] = jnp.full((16,), 1.0, F32)
        for i in range((_NPAD // _NS) // 16):
            dbuf[pl.ds(i * 16, 16)] = jnp.zeros((16,), F32)
        pltpu.sync_copy(dbuf, deg_sh.at[pl.ds(s * 640, 640)])
        plsc.subcore_barrier()

        @pl.when(c == 0)
        def _count():
            pltpu.sync_copy(dst_hbm.at[pl.ds(s * rows_per_sub, rows_per_sub), :], dst_v)

            def body(t, carry):
                pltpu.sync_copy(ones_v, deg_sh.at[dst_v.at[t]], add=True)
                return carry

            lax.fori_loop(0, rows_per_sub, body, 0)

        plsc.subcore_barrier()

        @pl.when(c == 0)
        def _finish():
            pltpu.sync_copy(deg_sh.at[pl.ds(s * 640, 640)], dbuf)
            pltpu.sync_copy(dbuf, deg_hbm.at[pl.ds(s * 640, 640)])

    return deg_kernel


def _make_agg_kernel():
    mesh = plsc.VectorSubcoreMesh(core_axis_name="c", subcore_axis_name="s")

    @functools.partial(
        pl.kernel,
        mesh=mesh,
        out_type=(
            jax.ShapeDtypeStruct((_ANROWS, _H), F32),
            jax.ShapeDtypeStruct((_ANROWS, _H), F32),
        ),
        scratch_types=[
            pltpu.VMEM((max(_AW0, _AW1), _CHUNK), jnp.int32),
            pltpu.VMEM((1, _CHUNK), jnp.int32),
            pltpu.VMEM((1, _CHUNK), jnp.int32),
            pltpu.VMEM((1, _CHUNK), jnp.int32),
            pltpu.VMEM((1, _CHUNK), jnp.int32),
            pltpu.VMEM((_CHUNK, _H), F32),
            pltpu.VMEM((_CHUNK, _H), F32),
            pltpu.VMEM_SHARED((_ANROWS, _H), F32),
            pltpu.SemaphoreType.DMA,
            pltpu.SemaphoreType.DMA,
        ],
    )
    def agg_kernel(g_hbm, packed_hbm, out0_hbm, out1_hbm,
                   pk_v, sr0, sr1, dr0, dr1, b0, b1, acc_sh, s0, s1):
        bufs = (b0, b1)
        sems = (s0, s1)
        srct = (sr0, sr1)
        dstt = (dr0, dr1)
        c = lax.axis_index("c")
        s = lax.axis_index("s")

        def unpack(t, r):
            for j in range(_CHUNK // 16):
                p = pk_v[t, pl.ds(j * 16, 16)]
                srct[r][0, pl.ds(j * 16, 16)] = lax.bitwise_and(
                    p, jnp.int32(0x3FFF))
                dstt[r][0, pl.ds(j * 16, 16)] = lax.shift_right_logical(p, 14)

        def run_edges(base, nrows):
            pltpu.sync_copy(
                packed_hbm.at[pl.ds(base, nrows), :], pk_v.at[pl.ds(0, nrows), :])
            unpack(0, 0)
            pltpu.async_copy(g_hbm.at[sr0.at[0]], b0, s0)

            def phase(t, p):
                np_ = (p + 1) % 2
                pltpu.make_async_copy(
                    g_hbm.at[srct[p].at[0]], bufs[p], sems[p]).wait()

                @pl.when(t + 1 < nrows)
                def _():
                    unpack(t + 1, np_)
                    pltpu.async_copy(
                        g_hbm.at[srct[np_].at[0]], bufs[np_], sems[np_])

                pltpu.sync_copy(bufs[p], acc_sh.at[dstt[p].at[0]], add=True)

            def body(j, carry):
                phase(2 * j, 0)
                phase(2 * j + 1, 1)
                return carry

            lax.fori_loop(0, nrows // 2, body, 0)

        def work(base, nrows, out_hbm):
            def zrow(i, carry):
                for j in range(_H // 16):
                    b0[i, pl.ds(j * 16, 16)] = jnp.zeros((16,), F32)
                return carry

            lax.fori_loop(0, _CHUNK, zrow, 0)
            for r0 in range(0, _ASUB, _CHUNK):
                sz = min(_CHUNK, _ASUB - r0)
                pltpu.sync_copy(
                    b0.at[pl.ds(0, sz), :],
                    acc_sh.at[pl.ds(s * _ASUB + r0, sz), :],
                )
            plsc.subcore_barrier()

            run_edges(base, nrows)
            plsc.subcore_barrier()

            for r0 in range(0, _ASUB, _CHUNK):
                sz = min(_CHUNK, _ASUB - r0)
                row = s * _ASUB + r0
                pltpu.sync_copy(acc_sh.at[pl.ds(row, sz), :], b0.at[pl.ds(0, sz), :])
                pltpu.sync_copy(b0.at[pl.ds(0, sz), :], out_hbm.at[pl.ds(row, sz), :])

        @pl.when(c == 0)
        def _w0():
            work(s * _AW0, _AW0, out0_hbm)

        @pl.when(c == 1)
        def _w1():
            work(_NS * _AW0 + s * _AW1, _AW1, out1_hbm)

    return agg_kernel


def _dinv(deg_ref):
    return lax.rsqrt(deg_ref[...] + 1.0)


def _tc1_body(x_ref, w_ref, deg_ref, g_ref):
    g_ref[...] = (
        jnp.dot(x_ref[...], w_ref[...], preferred_element_type=F32) * _dinv(deg_ref)
    )


def _tc2_body(a0_ref, a1_ref, g_ref, deg_ref, b_ref, w_ref, g2_ref, h_ref):
    dinv = _dinv(deg_ref)
    a = a0_ref[...] + a1_ref[...] + g_ref[...]
    h = jnp.maximum(a * dinv + b_ref[...], 0.0)
    h_ref[...] = h
    g2_ref[...] = jnp.dot(h, w_ref[...], preferred_element_type=F32) * dinv


def _tc3_body(a0_ref, a1_ref, g_ref, deg_ref, b_ref, res_ref, w_ref, g2_ref, h_ref):
    dinv = _dinv(deg_ref)
    a = a0_ref[...] + a1_ref[...] + g_ref[...]
    h = jnp.maximum(a * dinv + b_ref[...] + res_ref[...], 0.0)
    h_ref[...] = h
    g2_ref[...] = jnp.dot(h, w_ref[...], preferred_element_type=F32) * dinv


def _tc4_body(a0_ref, a1_ref, g_ref, deg_ref, b_ref, res_ref, fcw_ref, fcb_ref, o_ref):
    dinv = _dinv(deg_ref)
    a = a0_ref[...] + a1_ref[...] + g_ref[...]
    h = jnp.maximum(a * dinv + b_ref[...] + res_ref[...], 0.0)
    o = jnp.dot(h, fcw_ref[...], preferred_element_type=F32) + fcb_ref[...]
    o_ref[...] = jax.nn.sigmoid(o)


def _row_spec(i):
    return (i, 0)


def _full_spec(i):
    return (0, 0)


_ROWS = pl.BlockSpec((_R, _H), _row_spec)
_DINV = pl.BlockSpec((_R, 1), _row_spec)
_WFULL = pl.BlockSpec((_D, _H), _full_spec)
_BFULL = pl.BlockSpec((1, _H), _full_spec)


def _tc1(x, W, deg):
    return pl.pallas_call(
        _tc1_body,
        grid=(_GRID,),
        in_specs=[_ROWS, _WFULL, _DINV],
        out_specs=_ROWS,
        out_shape=jax.ShapeDtypeStruct((_N, _H), F32),
    )(x, W, deg)


def _tc2(acc, g, deg, b, W):
    return pl.pallas_call(
        _tc2_body,
        grid=(_GRID,),
        in_specs=[_ROWS, _ROWS, _ROWS, _DINV, _BFULL, _WFULL],
        out_specs=[_ROWS, _ROWS],
        out_shape=[
            jax.ShapeDtypeStruct((_N, _H), F32),
            jax.ShapeDtypeStruct((_N, _H), F32),
        ],
    )(acc[0], acc[1], g, deg, b, W)


def _tc3(acc, g, deg, b, res, W):
    return pl.pallas_call(
        _tc3_body,
        grid=(_GRID,),
        in_specs=[_ROWS, _ROWS, _ROWS, _DINV, _BFULL, _ROWS, _WFULL],
        out_specs=[_ROWS, _ROWS],
        out_shape=[
            jax.ShapeDtypeStruct((_N, _H), F32),
            jax.ShapeDtypeStruct((_N, _H), F32),
        ],
    )(acc[0], acc[1], g, deg, b, res, W)


def _tc4(acc, g, deg, b, res, fcW, fcb):
    return pl.pallas_call(
        _tc4_body,
        grid=(_GRID,),
        in_specs=[
            _ROWS,
            _ROWS,
            _ROWS,
            _DINV,
            _BFULL,
            _ROWS,
            pl.BlockSpec((_H, 1), _full_spec),
            pl.BlockSpec((1, 1), _full_spec),
        ],
        out_specs=pl.BlockSpec((_R, 1), _row_spec),
        out_shape=jax.ShapeDtypeStruct((_N, 1), F32),
    )(acc[0], acc[1], g, deg, b, res, fcW, fcb)


_deg_call = _make_deg_kernel()
_agg_call = _make_agg_kernel()


def kernel(x, edge_index, W1, b1, W2, b2, W3, b3, fc_W, fc_b):
    pad = _EPAD - _E
    src = jnp.concatenate([edge_index[0], jnp.zeros((pad,), jnp.int32)])
    dst = jnp.concatenate([edge_index[1], jnp.full((pad,), _N, jnp.int32)])
    packed2d = jnp.bitwise_or(src, jnp.left_shift(dst, 14)).reshape(_EROWS, _CHUNK)
    dst2d_deg = dst.reshape(_EROWS, _CHUNK)

    deg = _deg_call(dst2d_deg)[:_N].reshape(_N, 1)

    b1r = b1.reshape(1, _H)
    b2r = b2.reshape(1, _H)
    b3r = b3.reshape(1, _H)
    fcbr = fc_b.reshape(1, 1)

    g1 = _tc1(x, W1, deg)
    acc1 = _agg_call(g1, packed2d)
    g2, h1 = _tc2(acc1, g1, deg, b1r, W2)
    acc2 = _agg_call(g2, packed2d)
    g3, h2 = _tc3(acc2, g2, deg, b2r, h1, W3)
    acc3 = _agg_call(g3, packed2d)
    out = _tc4(acc3, g3, deg, b3r, h2, fc_W, fcbr)
    return out
